# baseline (speedup 1.0000x reference)
import functools

import jax
import jax.numpy as jnp
from jax import lax
from jax.experimental import pallas as pl
from jax.experimental.pallas import tpu as pltpu
from jax.experimental.pallas import tpu_sc as plsc

_N = 10000
_NP = 10240
_E = 320000
_S = 8
_DK = 16
_DM = 128
_TAU = 1.0

_NB = 1024
_CH = 1280
_CH2 = 2000


def _tc1_body(x_ref, w1_ref, b1_ref, w2_ref, b2_ref, o_ref):
    h = jnp.maximum(
        jnp.dot(x_ref[...], w1_ref[...], preferred_element_type=jnp.float32)
        + b1_ref[...], 0.0)
    o_ref[...] = (jnp.dot(h, w2_ref[...], preferred_element_type=jnp.float32)
                  + b2_ref[...])


def _tc1(x, W1, b1, W2, b2):
    nb = 1000
    grid = (_N // nb,)
    return pl.pallas_call(
        _tc1_body,
        grid=grid,
        in_specs=[
            pl.BlockSpec((nb, 256), lambda i: (i, 0)),
            pl.BlockSpec((256, 256), lambda i: (0, 0)),
            pl.BlockSpec((1, 256), lambda i: (0, 0)),
            pl.BlockSpec((256, 1), lambda i: (0, 0)),
            pl.BlockSpec((1, 1), lambda i: (0, 0)),
        ],
        out_specs=pl.BlockSpec((nb, 1), lambda i: (i, 0)),
        out_shape=jax.ShapeDtypeStruct((_N, 1), jnp.float32),
    )(x, W1, b1, W2, b2)


def _sc_mesh():
    return plsc.VectorSubcoreMesh(
        core_axis_name="c", subcore_axis_name="s", num_cores=2, num_subcores=16)


@functools.partial(
    pl.kernel,
    out_type=(jax.ShapeDtypeStruct((2, _DM, _NP), jnp.float32),
              jax.ShapeDtypeStruct((32, _NP), jnp.float32)),
    mesh=_sc_mesh(),
    compiler_params=pltpu.CompilerParams(needs_layout_passes=False),
    scratch_types=[
        pltpu.VMEM((_NP,), jnp.float32),
        pltpu.VMEM((32,), jnp.float32),
        [pltpu.VMEM((_NP,), jnp.float32)] * 8,
        pltpu.VMEM((_NP,), jnp.float32),
        pltpu.VMEM((2, _CH), jnp.int32),
        pltpu.VMEM((2, _CH), jnp.int32),
        pltpu.VMEM((2, 4, _CH), jnp.float32),
        pltpu.VMEM((_CH2,), jnp.int32),
        pltpu.SemaphoreType.DMA,
        pltpu.SemaphoreType.DMA,
    ],
)
def _sc1(ml_hbm, src_hbm, dst_hbm, ea4_hbm, cs_hbm, p_hbm, degp_hbm,
         ml_v, cs_v, acc, deg_v, src_v, dst_v, ea_v, dstd_v, sem0, sem1):
    cid = lax.axis_index("c")
    sid = lax.axis_index("s")
    wid = sid * 2 + cid
    g = wid // 2
    hh = wid % 2
    sp = g // 4
    q = g % 4

    pltpu.sync_copy(ml_hbm, ml_v.at[pl.ds(0, _N)])
    pltpu.sync_copy(cs_hbm.at[wid], cs_v)

    zf = jnp.zeros((16,), jnp.float32)

    def zbody(i, c):
        for k in range(8):
            acc[k][pl.ds(i * 16, 16)] = zf
        deg_v[pl.ds(i * 16, 16)] = zf
        return c
    lax.fori_loop(0, _NP // 16, zbody, 0)

    K0 = cs_v[pl.ds(0, 16)]
    K1 = cs_v[pl.ds(16, 16)]

    E2 = _E // 2
    NCHUNK = E2 // _CH
    base_e = hh * E2
    sems = (sem0, sem1)

    def _issue(chunk, b):
        off = base_e + chunk * _CH
        pltpu.async_copy(src_hbm.at[pl.ds(off, _CH)], src_v.at[b], sems[b])
        pltpu.async_copy(dst_hbm.at[pl.ds(off, _CH)], dst_v.at[b], sems[b])
        pltpu.async_copy(ea4_hbm.at[q, :, pl.ds(off, _CH)],
                         ea_v.at[b], sems[b])

    def _drain(b):
        pltpu.make_async_copy(src_hbm.at[pl.ds(0, _CH)], src_v.at[b],
                              sems[b]).wait()
        pltpu.make_async_copy(dst_hbm.at[pl.ds(0, _CH)], dst_v.at[b],
                              sems[b]).wait()
        pltpu.make_async_copy(ea4_hbm.at[0, :, pl.ds(0, _CH)],
                              ea_v.at[b], sems[b]).wait()

    def _process(b):
        @plsc.parallel_loop(0, _CH // 16, unroll=6)
        def _grp(j):
            s16 = src_v[b, pl.ds(j * 16, 16)]
            d16 = dst_v[b, pl.ds(j * 16, 16)]
            a = plsc.load_gather(ml_v, [s16])
            bb = plsc.load_gather(ml_v, [d16])
            t = jnp.exp(-jnp.maximum(a, bb))
            m0 = 1.0 / (1.0 + t * K0)
            m1 = 1.0 / (1.0 + t * K1)
            for k in range(4):
                eak = ea_v[b, k, pl.ds(j * 16, 16)]
                plsc.addupdate_scatter(acc[k], [d16], m0 * eak)
                plsc.addupdate_scatter(acc[4 + k], [d16], m1 * eak)

    _issue(0, 0)
    _issue(1, 1)

    def chunk_body(ci2, carry):
        ci = ci2 * 2
        for b in range(2):
            chunk = ci + b
            _drain(b)
            _process(b)

            @pl.when(chunk + 2 < NCHUNK)
            def _():
                _issue(chunk + 2, b)
        return carry
    lax.fori_loop(0, NCHUNK // 2, chunk_body, 0)
    if NCHUNK % 2:
        _drain(0)
        _process(0)

    ones16 = jnp.ones((16,), jnp.float32)
    EP = _E // 32

    def dchunk(ci, carry):
        off = wid * EP + ci * _CH2
        pltpu.sync_copy(dst_hbm.at[pl.ds(off, _CH2)], dstd_v)

        @plsc.parallel_loop(0, _CH2 // 16, unroll=4)
        def _dgrp(j):
            d16 = dstd_v[pl.ds(j * 16, 16)]
            plsc.addupdate_scatter(deg_v, [d16], ones16)
        return carry
    lax.fori_loop(0, EP // _CH2, dchunk, 0)

    descs = [pltpu.async_copy(
        acc[i * 4 + k], p_hbm.at[hh, (2 * sp + i) * 16 + q * 4 + k], sem0)
        for i in range(2) for k in range(4)]
    descs.append(pltpu.async_copy(deg_v, degp_hbm.at[wid], sem0))
    for d in descs:
        d.wait()


def _tc2_body(p_ref, degp_ref, wmT_ref, wuv_ref, o_ref):
    a = p_ref[0] + p_ref[1]
    deg = jnp.sum(degp_ref[...], axis=0)
    dinv = 1.0 / jnp.maximum(deg, 1.0)
    a = a * dinv[None, :]
    acc = jnp.zeros((_DM, a.shape[1]), jnp.float32)
    for s in range(_S):
        a_s = a[s * 16:(s + 1) * 16, :]
        acc = acc + jnp.maximum(
            jnp.dot(wmT_ref[...], a_s, preferred_element_type=jnp.float32), 0.0)
    o_ref[...] = jnp.dot(wuv_ref[...], acc * (1.0 / _S),
                         preferred_element_type=jnp.float32)


def _tc2(P, degP, WmT, Wuv):
    grid = (_NP // _NB,)
    return pl.pallas_call(
        _tc2_body,
        grid=grid,
        in_specs=[
            pl.BlockSpec((2, _DM, _NB), lambda i: (0, 0, i)),
            pl.BlockSpec((32, _NB), lambda i: (0, i)),
            pl.BlockSpec((_DM, _DK), lambda i: (0, 0)),
            pl.BlockSpec((2, _DM), lambda i: (0, 0)),
        ],
        out_specs=pl.BlockSpec((2, _NB), lambda i: (0, i)),
        out_shape=jax.ShapeDtypeStruct((2, _NP), jnp.float32),
    )(P, degP, WmT, Wuv)


@functools.partial(
    pl.kernel,
    out_type=jax.ShapeDtypeStruct((_E,), jnp.float32),
    mesh=_sc_mesh(),
    compiler_params=pltpu.CompilerParams(needs_layout_passes=False),
    scratch_types=[
        pltpu.VMEM((_NP,), jnp.float32),
        pltpu.VMEM((_NP,), jnp.float32),
        [pltpu.VMEM((_CH2,), jnp.int32)] * 2,
        [pltpu.VMEM((_CH2,), jnp.int32)] * 2,
        [pltpu.VMEM((_CH2,), jnp.float32)] * 2,
        pltpu.SemaphoreType.DMA,
        pltpu.SemaphoreType.DMA,
        pltpu.SemaphoreType.DMA,
        pltpu.SemaphoreType.DMA,
    ],
)
def _sc2(uvT_hbm, src_hbm, dst_hbm, out_hbm, u_v, v_v, src_v, dst_v, out_v,
         sem_i0, sem_i1, sem_o0, sem_o1):
    cid = lax.axis_index("c")
    sid = lax.axis_index("s")
    wid = sid * 2 + cid
    EP = _E // 32
    NCH = EP // _CH2
    sems_i = (sem_i0, sem_i1)
    sems_o = (sem_o0, sem_o1)
    base = wid * EP

    def _issue_in(ci, b):
        off = base + ci * _CH2
        pltpu.async_copy(src_hbm.at[pl.ds(off, _CH2)], src_v[b], sems_i[b])
        pltpu.async_copy(dst_hbm.at[pl.ds(off, _CH2)], dst_v[b], sems_i[b])

    def _drain_in(b):
        pltpu.make_async_copy(src_hbm.at[pl.ds(0, _CH2)], src_v[b],
                              sems_i[b]).wait()
        pltpu.make_async_copy(dst_hbm.at[pl.ds(0, _CH2)], dst_v[b],
                              sems_i[b]).wait()

    tdesc_u = pltpu.async_copy(uvT_hbm.at[0], u_v, sems_o[0])
    tdesc_v = pltpu.async_copy(uvT_hbm.at[1], v_v, sems_o[1])
    _issue_in(0, 0)
    _issue_in(1, 1)
    tdesc_u.wait()
    tdesc_v.wait()

    for ci in range(NCH):
        b = ci % 2
        _drain_in(b)
        if ci >= 2:
            pltpu.make_async_copy(out_v[b], out_hbm.at[pl.ds(0, _CH2)],
                                  sems_o[b]).wait()

        @plsc.parallel_loop(0, _CH2 // 16, unroll=4)
        def _grp(j):
            s16 = src_v[b][pl.ds(j * 16, 16)]
            d16 = dst_v[b][pl.ds(j * 16, 16)]
            out_v[b][pl.ds(j * 16, 16)] = (plsc.load_gather(u_v, [s16])
                                           + plsc.load_gather(v_v, [d16]))

        off = base + ci * _CH2
        pltpu.async_copy(out_v[b], out_hbm.at[pl.ds(off, _CH2)], sems_o[b])
        if ci + 2 < NCH:
            _issue_in(ci + 2, b)

    for b in range(2):
        pltpu.make_async_copy(out_v[b], out_hbm.at[pl.ds(0, _CH2)],
                              sems_o[b]).wait()


def kernel(node_embeddings, edge_attr, W1, b1, W2, b2, W_msg, W_pred, edge_index):
    cs = []
    for s in range(_S):
        eps = jax.random.uniform(jax.random.fold_in(jax.random.key(1), s), ())
        eps = jnp.clip(eps, 1e-6, 1.0 - 1e-6)
        cs.append((jnp.log2(eps) - jnp.log2(1.0 - eps)) / _TAU)
    cs8 = jnp.stack(cs).astype(jnp.float32)
    K8 = jnp.exp(-cs8)
    rows = jnp.repeat(K8.reshape(4, 2), 16, axis=1)
    cvec = jnp.repeat(rows, 8, axis=0)

    src = edge_index[0].astype(jnp.int32)
    dst = edge_index[1].astype(jnp.int32)
    ea4 = edge_attr.T.reshape(4, 4, _E)

    ml = _tc1(node_embeddings, W1, b1.reshape(1, -1), W2, b2.reshape(1, 1))
    P, degP = _sc1(ml.reshape(_N), src, dst, ea4, cvec)
    uvT = _tc2(P, degP, W_msg.T, W_pred.reshape(2, _DM))
    return _sc2(uvT, src, dst)


# speedup vs baseline: 1.0418x; 1.0418x over previous
import functools

import jax
import jax.numpy as jnp
from jax import lax
from jax.experimental import pallas as pl
from jax.experimental.pallas import tpu as pltpu
from jax.experimental.pallas import tpu_sc as plsc

_N = 10000
_NP = 10240
_E = 320000
_S = 8
_DK = 16
_DM = 128
_TAU = 1.0

_NB = 1024
_CH = 1280
_CH2 = 2000


def _tc1_body(x_ref, w1_ref, b1_ref, w2_ref, b2_ref, o_ref):
    h = jnp.maximum(
        jnp.dot(x_ref[...], w1_ref[...], preferred_element_type=jnp.float32)
        + b1_ref[...], 0.0)
    o_ref[...] = (jnp.dot(h, w2_ref[...], preferred_element_type=jnp.float32)
                  + b2_ref[...])


def _tc1(x, W1, b1, W2, b2):
    nb = 1000
    grid = (_N // nb,)
    return pl.pallas_call(
        _tc1_body,
        grid=grid,
        in_specs=[
            pl.BlockSpec((nb, 256), lambda i: (i, 0)),
            pl.BlockSpec((256, 256), lambda i: (0, 0)),
            pl.BlockSpec((1, 256), lambda i: (0, 0)),
            pl.BlockSpec((256, 1), lambda i: (0, 0)),
            pl.BlockSpec((1, 1), lambda i: (0, 0)),
        ],
        out_specs=pl.BlockSpec((nb, 1), lambda i: (i, 0)),
        out_shape=jax.ShapeDtypeStruct((_N, 1), jnp.float32),
    )(x, W1, b1, W2, b2)


def _sc_mesh():
    return plsc.VectorSubcoreMesh(
        core_axis_name="c", subcore_axis_name="s", num_cores=2, num_subcores=16)


@functools.partial(
    pl.kernel,
    out_type=(jax.ShapeDtypeStruct((2, _DM, _NP), jnp.float32),
              jax.ShapeDtypeStruct((32, _NP), jnp.float32)),
    mesh=_sc_mesh(),
    compiler_params=pltpu.CompilerParams(needs_layout_passes=False),
    scratch_types=[
        pltpu.VMEM((_NP,), jnp.float32),
        pltpu.VMEM((32,), jnp.float32),
        [pltpu.VMEM((_NP,), jnp.float32)] * 8,
        pltpu.VMEM((_NP,), jnp.float32),
        pltpu.VMEM((2, _CH), jnp.int32),
        pltpu.VMEM((2, _CH), jnp.int32),
        pltpu.VMEM((2, 4, _CH), jnp.float32),
        pltpu.VMEM((_CH2,), jnp.int32),
        pltpu.SemaphoreType.DMA,
        pltpu.SemaphoreType.DMA,
    ],
)
def _sc1(ml_hbm, src_hbm, dst_hbm, ea4_hbm, cs_hbm, p_hbm, degp_hbm,
         ml_v, cs_v, acc, deg_v, src_v, dst_v, ea_v, dstd_v, sem0, sem1):
    cid = lax.axis_index("c")
    sid = lax.axis_index("s")
    wid = sid * 2 + cid
    g = wid // 2
    hh = wid % 2
    sp = g // 4
    q = g % 4

    pltpu.sync_copy(ml_hbm, ml_v.at[pl.ds(0, _N)])
    pltpu.sync_copy(cs_hbm.at[wid], cs_v)

    zf = jnp.zeros((16,), jnp.float32)

    def zbody(i, c):
        for k in range(8):
            acc[k][pl.ds(i * 16, 16)] = zf
        deg_v[pl.ds(i * 16, 16)] = zf
        return c
    lax.fori_loop(0, _NP // 16, zbody, 0)

    K0 = cs_v[pl.ds(0, 16)]
    K1 = cs_v[pl.ds(16, 16)]

    E2 = _E // 2
    NCHUNK = E2 // _CH
    base_e = hh * E2
    sems = (sem0, sem1)

    def _issue(chunk, b):
        off = base_e + chunk * _CH
        pltpu.async_copy(src_hbm.at[pl.ds(off, _CH)], src_v.at[b], sems[b])
        pltpu.async_copy(dst_hbm.at[pl.ds(off, _CH)], dst_v.at[b], sems[b])
        pltpu.async_copy(ea4_hbm.at[q, :, pl.ds(off, _CH)],
                         ea_v.at[b], sems[b])

    def _drain(b):
        pltpu.make_async_copy(src_hbm.at[pl.ds(0, _CH)], src_v.at[b],
                              sems[b]).wait()
        pltpu.make_async_copy(dst_hbm.at[pl.ds(0, _CH)], dst_v.at[b],
                              sems[b]).wait()
        pltpu.make_async_copy(ea4_hbm.at[0, :, pl.ds(0, _CH)],
                              ea_v.at[b], sems[b]).wait()

    def _process(b):
        @plsc.parallel_loop(0, _CH // 16, unroll=4)
        def _grp(j):
            s16 = src_v[b, pl.ds(j * 16, 16)]
            d16 = dst_v[b, pl.ds(j * 16, 16)]
            a = plsc.load_gather(ml_v, [s16])
            bb = plsc.load_gather(ml_v, [d16])
            t = jnp.exp(-jnp.maximum(a, bb))
            m0 = 1.0 / (1.0 + t * K0)
            m1 = 1.0 / (1.0 + t * K1)
            for k in range(4):
                eak = ea_v[b, k, pl.ds(j * 16, 16)]
                plsc.addupdate_scatter(acc[k], [d16], m0 * eak)
                plsc.addupdate_scatter(acc[4 + k], [d16], m1 * eak)

    _issue(0, 0)
    _issue(1, 1)

    def chunk_body(ci2, carry):
        ci = ci2 * 2
        for b in range(2):
            chunk = ci + b
            _drain(b)
            _process(b)

            @pl.when(chunk + 2 < NCHUNK)
            def _():
                _issue(chunk + 2, b)
        return carry
    lax.fori_loop(0, NCHUNK // 2, chunk_body, 0)
    if NCHUNK % 2:
        _drain(0)
        _process(0)

    ones16 = jnp.ones((16,), jnp.float32)
    EP = _E // 32

    def dchunk(ci, carry):
        off = wid * EP + ci * _CH2
        pltpu.sync_copy(dst_hbm.at[pl.ds(off, _CH2)], dstd_v)

        @plsc.parallel_loop(0, _CH2 // 16, unroll=4)
        def _dgrp(j):
            d16 = dstd_v[pl.ds(j * 16, 16)]
            plsc.addupdate_scatter(deg_v, [d16], ones16)
        return carry
    lax.fori_loop(0, EP // _CH2, dchunk, 0)

    descs = [pltpu.async_copy(
        acc[i * 4 + k], p_hbm.at[hh, (2 * sp + i) * 16 + q * 4 + k], sem0)
        for i in range(2) for k in range(4)]
    descs.append(pltpu.async_copy(deg_v, degp_hbm.at[wid], sem0))
    for d in descs:
        d.wait()


def _tc2_body(p_ref, degp_ref, wmT_ref, wuv_ref, o_ref):
    a = p_ref[0] + p_ref[1]
    deg = jnp.sum(degp_ref[...], axis=0)
    dinv = 1.0 / jnp.maximum(deg, 1.0)
    a = a * dinv[None, :]
    acc = jnp.zeros((_DM, a.shape[1]), jnp.float32)
    for s in range(_S):
        a_s = a[s * 16:(s + 1) * 16, :]
        acc = acc + jnp.maximum(
            jnp.dot(wmT_ref[...], a_s, preferred_element_type=jnp.float32), 0.0)
    o_ref[...] = jnp.dot(wuv_ref[...], acc * (1.0 / _S),
                         preferred_element_type=jnp.float32)


def _tc2(P, degP, WmT, Wuv):
    grid = (_NP // _NB,)
    return pl.pallas_call(
        _tc2_body,
        grid=grid,
        in_specs=[
            pl.BlockSpec((2, _DM, _NB), lambda i: (0, 0, i)),
            pl.BlockSpec((32, _NB), lambda i: (0, i)),
            pl.BlockSpec((_DM, _DK), lambda i: (0, 0)),
            pl.BlockSpec((2, _DM), lambda i: (0, 0)),
        ],
        out_specs=pl.BlockSpec((2, _NB), lambda i: (0, i)),
        out_shape=jax.ShapeDtypeStruct((2, _NP), jnp.float32),
    )(P, degP, WmT, Wuv)


@functools.partial(
    pl.kernel,
    out_type=jax.ShapeDtypeStruct((_E,), jnp.float32),
    mesh=_sc_mesh(),
    compiler_params=pltpu.CompilerParams(needs_layout_passes=False),
    scratch_types=[
        pltpu.VMEM((_NP,), jnp.float32),
        pltpu.VMEM((_NP,), jnp.float32),
        [pltpu.VMEM((_CH2,), jnp.int32)] * 2,
        [pltpu.VMEM((_CH2,), jnp.int32)] * 2,
        [pltpu.VMEM((_CH2,), jnp.float32)] * 2,
        pltpu.SemaphoreType.DMA,
        pltpu.SemaphoreType.DMA,
        pltpu.SemaphoreType.DMA,
        pltpu.SemaphoreType.DMA,
    ],
)
def _sc2(uvT_hbm, src_hbm, dst_hbm, out_hbm, u_v, v_v, src_v, dst_v, out_v,
         sem_i0, sem_i1, sem_o0, sem_o1):
    cid = lax.axis_index("c")
    sid = lax.axis_index("s")
    wid = sid * 2 + cid
    EP = _E // 32
    NCH = EP // _CH2
    sems_i = (sem_i0, sem_i1)
    sems_o = (sem_o0, sem_o1)
    base = wid * EP

    def _issue_in(ci, b):
        off = base + ci * _CH2
        pltpu.async_copy(src_hbm.at[pl.ds(off, _CH2)], src_v[b], sems_i[b])
        pltpu.async_copy(dst_hbm.at[pl.ds(off, _CH2)], dst_v[b], sems_i[b])

    def _drain_in(b):
        pltpu.make_async_copy(src_hbm.at[pl.ds(0, _CH2)], src_v[b],
                              sems_i[b]).wait()
        pltpu.make_async_copy(dst_hbm.at[pl.ds(0, _CH2)], dst_v[b],
                              sems_i[b]).wait()

    tdesc_u = pltpu.async_copy(uvT_hbm.at[0], u_v, sems_o[0])
    tdesc_v = pltpu.async_copy(uvT_hbm.at[1], v_v, sems_o[1])
    _issue_in(0, 0)
    _issue_in(1, 1)
    tdesc_u.wait()
    tdesc_v.wait()

    for ci in range(NCH):
        b = ci % 2
        _drain_in(b)
        if ci >= 2:
            pltpu.make_async_copy(out_v[b], out_hbm.at[pl.ds(0, _CH2)],
                                  sems_o[b]).wait()

        @plsc.parallel_loop(0, _CH2 // 16, unroll=4)
        def _grp(j):
            s16 = src_v[b][pl.ds(j * 16, 16)]
            d16 = dst_v[b][pl.ds(j * 16, 16)]
            out_v[b][pl.ds(j * 16, 16)] = (plsc.load_gather(u_v, [s16])
                                           + plsc.load_gather(v_v, [d16]))

        off = base + ci * _CH2
        pltpu.async_copy(out_v[b], out_hbm.at[pl.ds(off, _CH2)], sems_o[b])
        if ci + 2 < NCH:
            _issue_in(ci + 2, b)

    for b in range(2):
        pltpu.make_async_copy(out_v[b], out_hbm.at[pl.ds(0, _CH2)],
                              sems_o[b]).wait()


def kernel(node_embeddings, edge_attr, W1, b1, W2, b2, W_msg, W_pred, edge_index):
    cs = []
    for s in range(_S):
        eps = jax.random.uniform(jax.random.fold_in(jax.random.key(1), s), ())
        eps = jnp.clip(eps, 1e-6, 1.0 - 1e-6)
        cs.append((jnp.log2(eps) - jnp.log2(1.0 - eps)) / _TAU)
    cs8 = jnp.stack(cs).astype(jnp.float32)
    K8 = jnp.exp(-cs8)
    rows = jnp.repeat(K8.reshape(4, 2), 16, axis=1)
    cvec = jnp.repeat(rows, 8, axis=0)

    src = edge_index[0].astype(jnp.int32)
    dst = edge_index[1].astype(jnp.int32)
    ea4 = edge_attr.T.reshape(4, 4, _E)

    ml = _tc1(node_embeddings, W1, b1.reshape(1, -1), W2, b2.reshape(1, 1))
    P, degP = _sc1(ml.reshape(_N), src, dst, ea4, cvec)
    uvT = _tc2(P, degP, W_msg.T, W_pred.reshape(2, _DM))
    return _sc2(uvT, src, dst)
